# baseline (speedup 1.0000x reference)
import functools

import jax
import jax.numpy as jnp
from jax import lax
from jax.experimental import pallas as pl
from jax.experimental.pallas import tpu as pltpu
from jax.experimental.pallas import tpu_sc as plsc

N = 10000
F = 128
FH = 64
NP = 10112
NB = NP // 128
E = 320000
RT = 80
ET = RT * 128
EP = 32 * ET
NC, NS = 2, 16
STRIPE = NP // NS
NBUF = 2
HDEPTH = 4

_mesh = plsc.VectorSubcoreMesh(core_axis_name="c", subcore_axis_name="s")


@functools.partial(
    pl.kernel,
    out_type=jax.ShapeDtypeStruct((NC, 16384), jnp.float32),
    mesh=_mesh,
    scratch_types=[
        pltpu.VMEM((RT, 128), jnp.int32),
        pltpu.VMEM((128,), jnp.float32),
        pltpu.VMEM_SHARED((128 * 128,), jnp.float32),
    ] + [pltpu.SemaphoreType.DMA] * HDEPTH,
)
def _deg_kernel(dst_hbm, zh_hbm, out_hbm, bin_v, ones_v, hist_sh, *hsems):
    c = lax.axis_index("c")
    s = lax.axis_index("s")
    wid = c * NS + s
    pltpu.sync_copy(dst_hbm.at[pl.ds(wid * RT, RT)], bin_v)
    for i in range(8):
        ones_v[pl.ds(i * 16, 16)] = jnp.ones((16,), jnp.float32)
    pltpu.sync_copy(
        zh_hbm.at[pl.ds(s * (16384 // NS), 16384 // NS)],
        hist_sh.at[pl.ds(s * (16384 // NS), 16384 // NS)],
    )
    plsc.subcore_barrier()
    for b in range(HDEPTH):
        pltpu.async_copy(ones_v, hist_sh.at[bin_v.at[b]], hsems[b], add=True)

    def body(g, carry):
        base = g * HDEPTH
        for b in range(HDEPTH):
            pltpu.make_async_copy(ones_v, hist_sh.at[bin_v.at[0]], hsems[b]).wait()
            jn = base + HDEPTH + b

            @pl.when(jn < RT)
            def _():
                pltpu.async_copy(ones_v, hist_sh.at[bin_v.at[jn]], hsems[b], add=True)

        return carry

    lax.fori_loop(0, RT // HDEPTH, body, 0)
    plsc.subcore_barrier()

    @pl.when(s == 0)
    def _():
        pltpu.sync_copy(hist_sh, out_hbm.at[c])


@functools.partial(
    pl.kernel,
    out_type=jax.ShapeDtypeStruct((2, NC, NP, FH), jnp.float32),
    mesh=_mesh,
    compiler_params=pltpu.CompilerParams(use_tc_tiling_on_sc=False),
    scratch_types=[
        pltpu.VMEM((RT, 128), jnp.int32),
        pltpu.VMEM((RT, 128), jnp.int32),
        pltpu.VMEM((NBUF, 128, FH), jnp.float32),
        pltpu.VMEM_SHARED((NP, FH), jnp.float32),
        pltpu.VMEM_SHARED((NP, FH), jnp.float32),
    ] + [pltpu.SemaphoreType.DMA] * (2 * NBUF),
)
def _agg_kernel(
    y0_hbm, y1_hbm, src_hbm, dst_hbm, zeros_hbm, out_hbm,
    src_v, dst_v, rows_v, ytab_sh, acc_sh, *sems
):
    gsem = sems[:NBUF]
    ssem = sems[NBUF:]
    c = lax.axis_index("c")
    s = lax.axis_index("s")
    wid = c * NS + s
    pltpu.sync_copy(src_hbm.at[pl.ds(wid * RT, RT)], src_v)
    pltpu.sync_copy(dst_hbm.at[pl.ds(wid * RT, RT)], dst_v)

    for p, y_hbm in ((0, y0_hbm), (1, y1_hbm)):
        pltpu.sync_copy(
            y_hbm.at[pl.ds(s * STRIPE, STRIPE)], ytab_sh.at[pl.ds(s * STRIPE, STRIPE)]
        )

        @pl.when(c == 0)
        def _():
            pltpu.sync_copy(
                y_hbm.at[pl.ds(s * STRIPE, STRIPE)], acc_sh.at[pl.ds(s * STRIPE, STRIPE)]
            )

        @pl.when(c == 1)
        def _():
            pltpu.sync_copy(
                zeros_hbm.at[pl.ds(s * STRIPE, STRIPE)],
                acc_sh.at[pl.ds(s * STRIPE, STRIPE)],
            )

        plsc.subcore_barrier()

        for b in range(NBUF):
            pltpu.async_copy(ytab_sh.at[src_v.at[b]], rows_v.at[b], gsem[b])

        def step(g, carry):
            base = g * NBUF
            for b in range(NBUF):
                j = base + b
                pltpu.make_async_copy(
                    ytab_sh.at[src_v.at[0]], rows_v.at[b], gsem[b]
                ).wait()
                pltpu.async_copy(rows_v.at[b], acc_sh.at[dst_v.at[j]], ssem[b], add=True)
            for b in range(NBUF):
                jn = base + NBUF + b

                @pl.when(jn < RT)
                def _():
                    pltpu.make_async_copy(
                        rows_v.at[b], acc_sh.at[dst_v.at[0]], ssem[b]
                    ).wait()
                    pltpu.async_copy(ytab_sh.at[src_v.at[jn]], rows_v.at[b], gsem[b])

            return carry

        lax.fori_loop(0, RT // NBUF, step, 0)
        for b in range(NBUF):
            pltpu.make_async_copy(rows_v.at[b], acc_sh.at[dst_v.at[0]], ssem[b]).wait()
        plsc.subcore_barrier()
        pltpu.sync_copy(
            acc_sh.at[pl.ds(s * STRIPE, STRIPE)],
            out_hbm.at[p, c, pl.ds(s * STRIPE, STRIPE)],
        )
        if p == 0:
            plsc.subcore_barrier()


def _mm_body(x_ref, w_ref, xw_ref):
    xw_ref[:N] = jnp.dot(x_ref[...], w_ref[...], preferred_element_type=jnp.float32)
    xw_ref[N:] = jnp.zeros((NP - N, F), jnp.float32)


_mm = pl.pallas_call(
    _mm_body,
    out_shape=[jax.ShapeDtypeStruct((NP, F), jnp.float32)],
)


def _dcol(h_ref):
    deg = h_ref[0] + h_ref[1] + 1.0
    d = lax.rsqrt(deg)
    return d[:NB][:, :, None]


def _scale_body(xw_ref, h_ref, y0_ref, y1_ref):
    y3 = xw_ref[...].reshape(NB, 128, F) * _dcol(h_ref)
    y = y3.reshape(NP, F)
    y0_ref[...] = y[:, :FH]
    y1_ref[...] = y[:, FH:]


_scale = pl.pallas_call(
    _scale_body,
    out_shape=[
        jax.ShapeDtypeStruct((NP, FH), jnp.float32),
        jax.ShapeDtypeStruct((NP, FH), jnp.float32),
    ],
)


def _fin_body(acc_ref, h_ref, b_ref, relu_ref, out_ref):
    dc = _dcol(h_ref)
    t0 = ((acc_ref[0, 0] + acc_ref[0, 1]).reshape(NB, 128, FH) * dc).reshape(NP, FH)
    t1 = ((acc_ref[1, 0] + acc_ref[1, 1]).reshape(NB, 128, FH) * dc).reshape(NP, FH)
    o0 = t0[:N] + b_ref[:, :FH]
    o1 = t1[:N] + b_ref[:, FH:]
    out_ref[:, :FH] = o0
    out_ref[:, FH:] = o1
    relu_ref[:, :FH] = jnp.maximum(o0, 0.0)
    relu_ref[:, FH:] = jnp.maximum(o1, 0.0)


_fin = pl.pallas_call(
    _fin_body,
    out_shape=[
        jax.ShapeDtypeStruct((N, F), jnp.float32),
        jax.ShapeDtypeStruct((N, F), jnp.float32),
    ],
)


def kernel(x, edge_index, W, b):
    fill = jnp.full((EP - E,), N, jnp.int32)
    src_p = jnp.concatenate([edge_index[0], fill]).reshape(EP // 128, 128)
    dst_p = jnp.concatenate([edge_index[1], fill]).reshape(EP // 128, 128)
    zeros_np = jnp.zeros((NP, FH), jnp.float32)
    zh = jnp.zeros((16384,), jnp.float32)

    hist = _deg_kernel(dst_p, zh).reshape(NC, 128, 128)
    (xw,) = _mm(x, W)
    y0, y1 = _scale(xw, hist)
    acc = _agg_kernel(y0, y1, src_p, dst_p, zeros_np)
    relu_o, o = _fin(acc, hist, b.reshape(1, F))
    return (relu_o, o)


# speedup vs baseline: 1.0499x; 1.0499x over previous
import functools

import jax
import jax.numpy as jnp
from jax import lax
from jax.experimental import pallas as pl
from jax.experimental.pallas import tpu as pltpu
from jax.experimental.pallas import tpu_sc as plsc

N = 10000
F = 128
FH = 64
NP = 10112
NB = NP // 128
E = 320000
RT = 80
ET = RT * 128
EP = 32 * ET
NC, NS = 2, 16
STRIPE = NP // NS
NBUF = 2
HDEPTH = 4

_mesh = plsc.VectorSubcoreMesh(core_axis_name="c", subcore_axis_name="s")


@functools.partial(
    pl.kernel,
    out_type=jax.ShapeDtypeStruct((NC, 16384), jnp.float32),
    mesh=_mesh,
    compiler_params=pltpu.CompilerParams(use_tc_tiling_on_sc=False),
    scratch_types=[
        pltpu.VMEM((RT, 128), jnp.int32),
        pltpu.VMEM((128,), jnp.float32),
        pltpu.VMEM_SHARED((128 * 128,), jnp.float32),
    ] + [pltpu.SemaphoreType.DMA] * HDEPTH,
)
def _deg_kernel(ei_hbm, zh_hbm, out_hbm, bin_v, ones_v, hist_sh, *hsems):
    c = lax.axis_index("c")
    s = lax.axis_index("s")
    wid = c * NS + s
    pltpu.sync_copy(ei_hbm.at[1, pl.ds(wid * RT, RT)], bin_v)
    for i in range(8):
        ones_v[pl.ds(i * 16, 16)] = jnp.ones((16,), jnp.float32)
    pltpu.sync_copy(
        zh_hbm.at[pl.ds(s * (16384 // NS), 16384 // NS)],
        hist_sh.at[pl.ds(s * (16384 // NS), 16384 // NS)],
    )
    plsc.subcore_barrier()
    for b in range(HDEPTH):
        pltpu.async_copy(ones_v, hist_sh.at[bin_v.at[b]], hsems[b], add=True)

    def body(g, carry):
        base = g * HDEPTH
        for b in range(HDEPTH):
            pltpu.make_async_copy(ones_v, hist_sh.at[bin_v.at[0]], hsems[b]).wait()
            jn = base + HDEPTH + b

            @pl.when(jn < RT)
            def _():
                pltpu.async_copy(ones_v, hist_sh.at[bin_v.at[jn]], hsems[b], add=True)

        return carry

    lax.fori_loop(0, RT // HDEPTH, body, 0)
    plsc.subcore_barrier()

    @pl.when(s == 0)
    def _():
        pltpu.sync_copy(hist_sh, out_hbm.at[c])


@functools.partial(
    pl.kernel,
    out_type=jax.ShapeDtypeStruct((2, NC, NP, FH), jnp.float32),
    mesh=_mesh,
    compiler_params=pltpu.CompilerParams(use_tc_tiling_on_sc=False),
    scratch_types=[
        pltpu.VMEM((RT, 128), jnp.int32),
        pltpu.VMEM((RT, 128), jnp.int32),
        pltpu.VMEM((NBUF, 128, FH), jnp.float32),
        pltpu.VMEM_SHARED((NP, FH), jnp.float32),
        pltpu.VMEM_SHARED((NP, FH), jnp.float32),
    ] + [pltpu.SemaphoreType.DMA] * (2 * NBUF),
)
def _agg_kernel(
    y0_hbm, y1_hbm, ei_hbm, zeros_hbm, out_hbm,
    src_v, dst_v, rows_v, ytab_sh, acc_sh, *sems
):
    gsem = sems[:NBUF]
    ssem = sems[NBUF:]
    c = lax.axis_index("c")
    s = lax.axis_index("s")
    wid = c * NS + s
    pltpu.sync_copy(ei_hbm.at[0, pl.ds(wid * RT, RT)], src_v)
    pltpu.sync_copy(ei_hbm.at[1, pl.ds(wid * RT, RT)], dst_v)

    for p, y_hbm in ((0, y0_hbm), (1, y1_hbm)):
        pltpu.sync_copy(
            y_hbm.at[pl.ds(s * STRIPE, STRIPE)], ytab_sh.at[pl.ds(s * STRIPE, STRIPE)]
        )

        @pl.when(c == 0)
        def _():
            pltpu.sync_copy(
                y_hbm.at[pl.ds(s * STRIPE, STRIPE)], acc_sh.at[pl.ds(s * STRIPE, STRIPE)]
            )

        @pl.when(c == 1)
        def _():
            pltpu.sync_copy(
                zeros_hbm.at[pl.ds(s * STRIPE, STRIPE)],
                acc_sh.at[pl.ds(s * STRIPE, STRIPE)],
            )

        plsc.subcore_barrier()

        for b in range(NBUF):
            pltpu.async_copy(ytab_sh.at[src_v.at[b]], rows_v.at[b], gsem[b])

        def step(g, carry):
            base = g * NBUF
            for b in range(NBUF):
                j = base + b
                pltpu.make_async_copy(
                    ytab_sh.at[src_v.at[0]], rows_v.at[b], gsem[b]
                ).wait()
                pltpu.async_copy(rows_v.at[b], acc_sh.at[dst_v.at[j]], ssem[b], add=True)
            for b in range(NBUF):
                jn = base + NBUF + b

                @pl.when(jn < RT)
                def _():
                    pltpu.make_async_copy(
                        rows_v.at[b], acc_sh.at[dst_v.at[0]], ssem[b]
                    ).wait()
                    pltpu.async_copy(ytab_sh.at[src_v.at[jn]], rows_v.at[b], gsem[b])

            return carry

        lax.fori_loop(0, RT // NBUF, step, 0)
        for b in range(NBUF):
            pltpu.make_async_copy(rows_v.at[b], acc_sh.at[dst_v.at[0]], ssem[b]).wait()
        plsc.subcore_barrier()
        pltpu.sync_copy(
            acc_sh.at[pl.ds(s * STRIPE, STRIPE)],
            out_hbm.at[p, c, pl.ds(s * STRIPE, STRIPE)],
        )
        if p == 0:
            plsc.subcore_barrier()


def _mm_body(x_ref, w_ref, xw_ref):
    xw_ref[:N] = jnp.dot(x_ref[...], w_ref[...], preferred_element_type=jnp.float32)
    xw_ref[N:] = jnp.zeros((NP - N, F), jnp.float32)


_mm = pl.pallas_call(
    _mm_body,
    out_shape=[jax.ShapeDtypeStruct((NP, F), jnp.float32)],
)


def _dcol(h_ref):
    deg = h_ref[0] + h_ref[1] + 1.0
    d = lax.rsqrt(deg)
    return d[:NB][:, :, None]


def _scale_body(xw_ref, h_ref, y0_ref, y1_ref):
    y3 = xw_ref[...].reshape(NB, 128, F) * _dcol(h_ref)
    y = y3.reshape(NP, F)
    y0_ref[...] = y[:, :FH]
    y1_ref[...] = y[:, FH:]


_scale = pl.pallas_call(
    _scale_body,
    out_shape=[
        jax.ShapeDtypeStruct((NP, FH), jnp.float32),
        jax.ShapeDtypeStruct((NP, FH), jnp.float32),
    ],
)


def _fin_body(acc_ref, h_ref, b_ref, relu_ref, out_ref):
    dc = _dcol(h_ref)
    t0 = ((acc_ref[0, 0] + acc_ref[0, 1]).reshape(NB, 128, FH) * dc).reshape(NP, FH)
    t1 = ((acc_ref[1, 0] + acc_ref[1, 1]).reshape(NB, 128, FH) * dc).reshape(NP, FH)
    o0 = t0[:N] + b_ref[:, :FH]
    o1 = t1[:N] + b_ref[:, FH:]
    out_ref[:, :FH] = o0
    out_ref[:, FH:] = o1
    relu_ref[:, :FH] = jnp.maximum(o0, 0.0)
    relu_ref[:, FH:] = jnp.maximum(o1, 0.0)


_fin = pl.pallas_call(
    _fin_body,
    out_shape=[
        jax.ShapeDtypeStruct((N, F), jnp.float32),
        jax.ShapeDtypeStruct((N, F), jnp.float32),
    ],
)


def kernel(x, edge_index, W, b):
    ei_p = jnp.pad(edge_index, ((0, 0), (0, EP - E)), constant_values=N)
    ei_p = ei_p.reshape(2, EP // 128, 128)
    zeros_np = jnp.zeros((NP, FH), jnp.float32)
    zh = jnp.zeros((16384,), jnp.float32)

    hist = _deg_kernel(ei_p, zh).reshape(NC, 128, 128)
    (xw,) = _mm(x, W)
    y0, y1 = _scale(xw, hist)
    acc = _agg_kernel(y0, y1, ei_p, zeros_np)
    relu_o, o = _fin(acc, hist, b.reshape(1, F))
    return (relu_o, o)
